# Initial kernel scaffold; baseline (speedup 1.0000x reference)
import jax
import jax.numpy as jnp
from jax.experimental import pallas as pl


def kernel(input_ids_a, input_ids_b, mix_idxes, mix_ratios, table):
    raise NotImplementedError("write your pallas kernel here")


# speedup vs baseline: 2.7690x; 2.7690x over previous
import functools

import jax
import jax.numpy as jnp
from jax import lax
from jax.experimental import pallas as pl
from jax.experimental.pallas import tpu as pltpu
from jax.experimental.pallas import tpu_sc as plsc

B = 4096
L = 200
D = 32
NMIX = 50
NC = 2
NS = 16
NW = NC * NS
RPW = B // NW
CB = 8
NCHUNK = RPW // CB
AROWS = CB * L
APAD = 1664
BSLOT = CB * NMIX
BPAD = 512
ISTREAM = 128
NT = BSLOT // 16


def _sc_embed_mix(table, ids_a_flat, ids_b_flat, mix_i_flat, mix_r_flat):
    mesh = plsc.VectorSubcoreMesh(core_axis_name="c", subcore_axis_name="s")

    @functools.partial(
        pl.kernel,
        mesh=mesh,
        out_type=jax.ShapeDtypeStruct((B * L, D), jnp.float32),
        compiler_params=pltpu.CompilerParams(
            needs_layout_passes=False, use_tc_tiling_on_sc=False),
        scratch_types=[
            pltpu.VMEM((APAD,), jnp.int32),
            pltpu.VMEM((APAD, D), jnp.float32),
            pltpu.VMEM((AROWS,), jnp.int32),
            pltpu.VMEM((BSLOT,), jnp.int32),
            pltpu.VMEM((BSLOT,), jnp.float32),
            pltpu.VMEM((BPAD,), jnp.int32),
            pltpu.VMEM((BPAD, D), jnp.float32),
            pltpu.VMEM((BSLOT,), jnp.int32),
            pltpu.VMEM((NMIX * 256,), jnp.float32),
            pltpu.SemaphoreType.DMA,
            pltpu.SemaphoreType.DMA,
        ],
    )
    def k(table_h, ida_h, idb_h, mixi_h, mixr_h, out_h,
          ida_v, arows_v, idbrows_v, mixi_v, mixr_v, idbl_v, brows_v,
          offs_v, mbuf_v, sema, semb):
        wid = lax.axis_index("s") * NC + lax.axis_index("c")
        iota = lax.iota(jnp.int32, 16)
        lbv16 = iota & 7
        halfv16 = (iota >> 3) * 16
        lb50 = lbv16 * 50
        zeros16 = jnp.zeros((16,), jnp.int32)

        def chunk_body(g, carry):
            row0 = wid * RPW + g * CB
            pltpu.sync_copy(ida_h.at[pl.ds(row0 * L, AROWS)],
                            ida_v.at[pl.ds(0, AROWS)])
            for t in range((APAD - AROWS) // 16):
                ida_v[pl.ds(AROWS + t * 16, 16)] = zeros16
            adescs = [
                pltpu.async_copy(
                    table_h.at[ida_v.at[pl.ds(j * ISTREAM, ISTREAM)]],
                    arows_v.at[pl.ds(j * ISTREAM, ISTREAM)], sema)
                for j in range(APAD // ISTREAM)
            ]
            pltpu.sync_copy(idb_h.at[pl.ds(row0 * L, AROWS)], idbrows_v)
            pltpu.sync_copy(mixi_h.at[pl.ds(row0 * NMIX, BSLOT)], mixi_v)
            pltpu.sync_copy(mixr_h.at[pl.ds(row0 * NMIX, BSLOT)], mixr_v)

            def build_slot(t, c):
                sv = t * 16 + iota
                jjv = sv >> 3
                lbv = sv & 7
                lvec = plsc.load_gather(mixi_v, [lbv * 50 + jjv])
                offv = lbv * 200 + lvec
                idbv = plsc.load_gather(idbrows_v, [offv])
                offs_v[pl.ds(t * 16, 16)] = offv
                idbl_v[pl.ds(t * 16, 16)] = idbv
                return c
            lax.fori_loop(0, NT, build_slot, 0)
            for t in range((BPAD - BSLOT) // 16):
                idbl_v[pl.ds(BSLOT + t * 16, 16)] = zeros16
            bdescs = [
                pltpu.async_copy(
                    table_h.at[idbl_v.at[pl.ds(j * ISTREAM, ISTREAM)]],
                    brows_v.at[pl.ds(j * ISTREAM, ISTREAM)], semb)
                for j in range(BPAD // ISTREAM)
            ]
            for dsc in adescs:
                dsc.wait()
            for dsc in bdescs:
                dsc.wait()

            def mix_blend(jj, c):
                sidx = jj * 8 + lbv16
                offg = plsc.load_gather(offs_v, [sidx])
                rg = plsc.load_gather(mixr_v, [lb50 + jj])
                mb = jj * 256
                for d16 in range(16):
                    colv = halfv16 + d16
                    va = plsc.load_gather(arows_v, [offg, colv])
                    vb = plsc.load_gather(brows_v, [sidx, colv])
                    mbuf_v[pl.ds(mb + d16 * 16, 16)] = vb + rg * (va - vb)
                return c
            lax.fori_loop(0, NMIX, mix_blend, 0)

            def mix_write(jj, c):
                offg = plsc.load_gather(offs_v, [jj * 8 + lbv16])
                mb = jj * 256
                for d16 in range(16):
                    m = mbuf_v[pl.ds(mb + d16 * 16, 16)]
                    plsc.store_scatter(arows_v, [offg, halfv16 + d16], m)
                return c
            lax.fori_loop(0, NMIX, mix_write, 0)

            pltpu.sync_copy(arows_v.at[pl.ds(0, AROWS)],
                            out_h.at[pl.ds(row0 * L, AROWS)])
            return carry

        lax.fori_loop(0, NCHUNK, chunk_body, 0)

    return k(table, ids_a_flat, ids_b_flat, mix_i_flat, mix_r_flat)


def kernel(input_ids_a, input_ids_b, mix_idxes, mix_ratios, table):
    ids_a = input_ids_a.astype(jnp.int32).reshape(B * L)
    ids_b = input_ids_b.astype(jnp.int32).reshape(B * L)
    mix_i = mix_idxes.astype(jnp.int32).reshape(B * NMIX)
    mix_r = mix_ratios.astype(jnp.float32).reshape(B * NMIX)
    out = _sc_embed_mix(table, ids_a, ids_b, mix_i, mix_r)
    return out.reshape(B, L, D)


# speedup vs baseline: 5.0903x; 1.8383x over previous
import functools

import jax
import jax.numpy as jnp
from jax import lax
from jax.experimental import pallas as pl
from jax.experimental.pallas import tpu as pltpu
from jax.experimental.pallas import tpu_sc as plsc

B = 4096
L = 200
D = 32
NMIX = 50
NC = 2
NS = 16
NW = NC * NS
RPW = B // NW
CB = 4
NCHUNK = RPW // CB
AROWS = CB * L
BSLOT = CB * NMIX
NT = (BSLOT + 15) // 16
BPAD = NT * 16
OFFPAD = BPAD + 16
ISTREAM = 128
GROUP = 6


def _streams(n):
    out, base = [], 0
    while base < n:
        w = min(ISTREAM, n - base)
        out.append((base, w))
        base += w
    return out


ASTREAMS = _streams(AROWS)
BSTREAMS = _streams(BPAD)


def _sc_embed_mix(table, ids_a_flat, ids_b_flat, mix_i_flat, mix_r_flat):
    mesh = plsc.VectorSubcoreMesh(core_axis_name="c", subcore_axis_name="s")

    scratch = (
        [pltpu.VMEM((AROWS,), jnp.int32) for _ in range(2)]
        + [pltpu.VMEM((AROWS,), jnp.int32) for _ in range(2)]
        + [pltpu.VMEM((BSLOT,), jnp.int32) for _ in range(2)]
        + [pltpu.VMEM((BSLOT,), jnp.float32) for _ in range(2)]
        + [pltpu.VMEM((OFFPAD,), jnp.int32) for _ in range(2)]
        + [pltpu.VMEM((OFFPAD,), jnp.float32) for _ in range(2)]
        + [pltpu.VMEM((BPAD,), jnp.int32) for _ in range(2)]
        + [pltpu.VMEM((BPAD, D), jnp.float32) for _ in range(2)]
        + [pltpu.VMEM((AROWS, D), jnp.float32) for _ in range(3)]
        + [pltpu.SemaphoreType.DMA for _ in range(10)]
    )

    @functools.partial(
        pl.kernel,
        mesh=mesh,
        out_type=jax.ShapeDtypeStruct((B * L, D), jnp.float32),
        compiler_params=pltpu.CompilerParams(
            needs_layout_passes=False, use_tc_tiling_on_sc=False),
        scratch_types=scratch,
    )
    def k(table_h, ida_h, idb_h, mixi_h, mixr_h, out_h, *sc):
        ida = sc[0:2]
        idbrows = sc[2:4]
        mixi = sc[4:6]
        mixr = sc[6:8]
        offs = sc[8:10]
        rbuf = sc[10:12]
        idbl = sc[12:14]
        brows = sc[14:16]
        arows = sc[16:19]
        sem_idx = sc[19:21]
        sem_b = sc[21:23]
        sem_a = sc[23:26]
        sem_wb = sc[26:29]

        wid = lax.axis_index("s") * NC + lax.axis_index("c")
        iota = lax.iota(jnp.int32, 16)

        def r0_of(c):
            return wid * RPW + c * CB

        def stage_idx(c, co):
            p = co % 2
            r0 = r0_of(c)
            pltpu.async_copy(ida_h.at[pl.ds(r0 * L, AROWS)], ida[p],
                             sem_idx[p])
            pltpu.async_copy(idb_h.at[pl.ds(r0 * L, AROWS)], idbrows[p],
                             sem_idx[p])
            pltpu.async_copy(mixi_h.at[pl.ds(r0 * NMIX, BSLOT)], mixi[p],
                             sem_idx[p])
            pltpu.async_copy(mixr_h.at[pl.ds(r0 * NMIX, BSLOT)], mixr[p],
                             sem_idx[p])

        def wait_idx(co):
            p = co % 2
            pltpu.make_async_copy(ida_h.at[pl.ds(0, AROWS)], ida[p],
                                  sem_idx[p]).wait()
            pltpu.make_async_copy(idb_h.at[pl.ds(0, AROWS)], idbrows[p],
                                  sem_idx[p]).wait()
            pltpu.make_async_copy(mixi_h.at[pl.ds(0, BSLOT)], mixi[p],
                                  sem_idx[p]).wait()
            pltpu.make_async_copy(mixr_h.at[pl.ds(0, BSLOT)], mixr[p],
                                  sem_idx[p]).wait()

        def build(co):
            p = co % 2

            def body(t, carry):
                sv = jnp.minimum(t * 16 + iota, BSLOT - 1)
                jjv = sv >> 2
                lbv = sv & 3
                midx = lbv * NMIX + jjv
                lvec = plsc.load_gather(mixi[p], [midx])
                rvec = plsc.load_gather(mixr[p], [midx])
                offv = lbv * L + lvec
                idbv = plsc.load_gather(idbrows[p], [offv])
                offs[p][pl.ds(t * 16, 16)] = offv
                rbuf[p][pl.ds(t * 16, 16)] = rvec
                idbl[p][pl.ds(t * 16, 16)] = idbv
                return carry

            lax.fori_loop(0, NT, body, 0)

        def fire_gathers(co):
            p = co % 2
            q = co % 3
            for base, w in ASTREAMS:
                pltpu.async_copy(table_h.at[ida[p].at[pl.ds(base, w)]],
                                 arows[q].at[pl.ds(base, w)], sem_a[q])
            for base, w in BSTREAMS:
                pltpu.async_copy(table_h.at[idbl[p].at[pl.ds(base, w)]],
                                 brows[p].at[pl.ds(base, w)], sem_b[p])

        def wait_gathers(co):
            p = co % 2
            q = co % 3
            pltpu.make_async_copy(table_h.at[pl.ds(0, AROWS)], arows[q],
                                  sem_a[q]).wait()
            pltpu.make_async_copy(table_h.at[pl.ds(0, BPAD)], brows[p],
                                  sem_b[p]).wait()

        def compute(co):
            p = co % 2
            q = co % 3

            def blend(jj, carry):
                offv = offs[p][pl.ds(jj * CB, 16)]
                rv = rbuf[p][pl.ds(jj * CB, 16)]
                for lb in range(CB):
                    off = offv[lb]
                    rr = rv[lb]
                    s = jj * CB + lb
                    for h in range(2):
                        va = arows[q][off, pl.ds(h * 16, 16)]
                        vb = brows[p][s, pl.ds(h * 16, 16)]
                        brows[p][s, pl.ds(h * 16, 16)] = vb + rr * (va - vb)
                return carry

            lax.fori_loop(0, NMIX, blend, 0)

            def write(jj, carry):
                offv = offs[p][pl.ds(jj * CB, 16)]
                for lb in range(CB):
                    off = offv[lb]
                    s = jj * CB + lb
                    for h in range(2):
                        arows[q][off, pl.ds(h * 16, 16)] = \
                            brows[p][s, pl.ds(h * 16, 16)]
                return carry

            lax.fori_loop(0, NMIX, write, 0)

        def fire_writeback(c, co):
            q = co % 3
            pltpu.async_copy(arows[q], out_h.at[pl.ds(r0_of(c) * L, AROWS)],
                             sem_wb[q])

        def wait_writeback(co):
            q = co % 3
            pltpu.make_async_copy(arows[q], out_h.at[pl.ds(0, AROWS)],
                                  sem_wb[q]).wait()

        def step(c, co):
            wait_idx(co)
            build(co)
            wait_writeback(co)
            fire_gathers(co)
            wait_gathers(co - 1)
            stage_idx(c + 1, co + 1)
            compute(co - 1)
            fire_writeback(c - 1, co - 1)

        stage_idx(0, 0)
        wait_idx(0)
        build(0)
        fire_gathers(0)
        stage_idx(1, 1)
        for c in range(1, GROUP):
            wait_idx(c)
            build(c)
            if c >= 3:
                wait_writeback(c)
            fire_gathers(c)
            wait_gathers(c - 1)
            stage_idx(c + 1, c + 1)
            compute(c - 1)
            fire_writeback(c - 1, c - 1)

        def group_body(i, carry):
            base = i * GROUP
            for o in range(GROUP):
                step(base + o, GROUP + o)
            return carry

        lax.fori_loop(1, 1 + (NCHUNK - GROUP - 2) // GROUP, group_body, 0)

        for c in range(NCHUNK - 2, NCHUNK):
            wait_idx(c)
            build(c)
            wait_writeback(c)
            fire_gathers(c)
            wait_gathers(c - 1)
            if c + 1 < NCHUNK:
                stage_idx(c + 1, c + 1)
            compute(c - 1)
            fire_writeback(c - 1, c - 1)
        c = NCHUNK - 1
        wait_gathers(c)
        compute(c)
        fire_writeback(c, c)
        for cc in range(NCHUNK - 3, NCHUNK):
            wait_writeback(cc)

    return k(table, ids_a_flat, ids_b_flat, mix_i_flat, mix_r_flat)


def kernel(input_ids_a, input_ids_b, mix_idxes, mix_ratios, table):
    ids_a = input_ids_a.astype(jnp.int32).reshape(B * L)
    ids_b = input_ids_b.astype(jnp.int32).reshape(B * L)
    mix_i = mix_idxes.astype(jnp.int32).reshape(B * NMIX)
    mix_r = mix_ratios.astype(jnp.float32).reshape(B * NMIX)
    out = _sc_embed_mix(table, ids_a, ids_b, mix_i, mix_r)
    return out.reshape(B, L, D)
